# baseline (speedup 1.0000x reference)
import functools

import jax
import jax.numpy as jnp
from jax import lax
from jax.experimental import pallas as pl
from jax.experimental.pallas import tpu as pltpu
from jax.experimental.pallas import tpu_sc as plsc

_SEQ = 200
_BATCH = 4096
_EMBED = 64
_B = _SEQ * _BATCH

_NC, _NS = 2, 16
_NW = _NC * _NS
_BPW = _B // _NW
_G = 128
_NG = 4
_NROW = _BPW // _G
_NCHUNK = _NROW // _NG

_mesh = plsc.VectorSubcoreMesh(core_axis_name="c", subcore_axis_name="s")


@functools.partial(
    pl.kernel,
    mesh=_mesh,
    out_type=jax.ShapeDtypeStruct((_NW, _NROW, _G, _EMBED), jnp.float32),
    scratch_types=[
        pltpu.VMEM((_NG, _G), jnp.int32),
        pltpu.VMEM((_NG, _G, _EMBED), jnp.float32),
        pltpu.SemaphoreType.DMA,
    ],
    compiler_params=pltpu.CompilerParams(use_tc_tiling_on_sc=False),
)
def _embed_gather(idx_hbm, table_hbm, out_hbm, idx_v, rows_v, sem):
    wid = lax.axis_index("s") * _NC + lax.axis_index("c")

    def chunk_body(ci, carry):
        row0 = ci * _NG
        pltpu.sync_copy(idx_hbm.at[wid, pl.ds(row0, _NG)], idx_v)
        copies = [
            pltpu.async_copy(table_hbm.at[idx_v.at[j]], rows_v.at[j], sem)
            for j in range(_NG)
        ]
        for c in copies:
            c.wait()
        pltpu.sync_copy(rows_v, out_hbm.at[wid, pl.ds(row0, _NG)])
        return carry

    lax.fori_loop(0, _NCHUNK, chunk_body, 0)


def kernel(indices, table):
    idx = indices.astype(jnp.int32).reshape(_NW, _NROW, _G)
    out = _embed_gather(idx, table)
    return out.reshape(_SEQ, _BATCH, _EMBED)


# speedup vs baseline: 1.0440x; 1.0440x over previous
import functools

import jax
import jax.numpy as jnp
from jax import lax
from jax.experimental import pallas as pl
from jax.experimental.pallas import tpu as pltpu
from jax.experimental.pallas import tpu_sc as plsc

_SEQ = 200
_BATCH = 4096
_EMBED = 64
_B = _SEQ * _BATCH

_NC, _NS = 2, 16
_NW = _NC * _NS
_BPW = _B // _NW
_G = 128
_NG = 4
_NROW = _BPW // _G
_NCHUNK = _NROW // _NG
_NBUF = 2
_NGROUP = _NCHUNK // _NBUF

_mesh = plsc.VectorSubcoreMesh(core_axis_name="c", subcore_axis_name="s")


@functools.partial(
    pl.kernel,
    mesh=_mesh,
    out_type=jax.ShapeDtypeStruct((_NW, _NROW, _G, _EMBED), jnp.float32),
    scratch_types=[
        pltpu.VMEM((_NROW, _G), jnp.int32),
        pltpu.VMEM((_NBUF, _NG, _G, _EMBED), jnp.float32),
        pltpu.SemaphoreType.DMA,
        pltpu.SemaphoreType.DMA,
        pltpu.SemaphoreType.DMA,
        pltpu.SemaphoreType.DMA,
    ],
    compiler_params=pltpu.CompilerParams(use_tc_tiling_on_sc=False),
)
def _embed_gather(idx_hbm, table_hbm, out_hbm, idx_v, rows_v, g0, g1, w0, w1):
    gsem = [g0, g1]
    wsem = [w0, w1]
    wid = lax.axis_index("s") * _NC + lax.axis_index("c")

    def gstart(ci, b):
        row0 = ci * _NG
        for j in range(_NG):
            pltpu.async_copy(
                table_hbm.at[idx_v.at[row0 + j]], rows_v.at[b, j], gsem[b])

    def gwait(ci, b):
        row0 = ci * _NG
        for j in range(_NG):
            pltpu.make_async_copy(
                table_hbm.at[idx_v.at[row0 + j]], rows_v.at[b, j],
                gsem[b]).wait()

    def wstart(ci, b):
        pltpu.async_copy(
            rows_v.at[b], out_hbm.at[wid, pl.ds(ci * _NG, _NG)], wsem[b])

    def wwait(ci, b):
        pltpu.make_async_copy(
            rows_v.at[b], out_hbm.at[wid, pl.ds(ci * _NG, _NG)],
            wsem[b]).wait()

    pltpu.sync_copy(idx_hbm.at[wid], idx_v)

    for b in range(_NBUF):
        gstart(b, b)

    def group(g, carry):
        ci0 = g * _NBUF
        for b in range(_NBUF):
            gwait(ci0 + b, b)
            wstart(ci0 + b, b)
        for b in range(_NBUF):
            wwait(ci0 + b, b)
            gstart(ci0 + _NBUF + b, b)
        return carry

    lax.fori_loop(0, _NGROUP - 1, group, 0)

    ci0 = (_NGROUP - 1) * _NBUF
    for b in range(_NBUF):
        gwait(ci0 + b, b)
        wstart(ci0 + b, b)
    for b in range(_NBUF):
        wwait(ci0 + b, b)


def kernel(indices, table):
    idx = indices.astype(jnp.int32).reshape(_NW, _NROW, _G)
    out = _embed_gather(idx, table)
    return out.reshape(_SEQ, _BATCH, _EMBED)
